# baseline (speedup 1.0000x reference)
import functools

import jax
import jax.numpy as jnp
from jax import lax
from jax.experimental import pallas as pl
from jax.experimental.pallas import tpu as pltpu
from jax.experimental.pallas import tpu_sc as plsc

F = 1025
T = 256
B = 2
MAXP = 5
MARGIN = 3
PWR = 0.1
LLIM = F - (MARGIN + 1)


def _tc_body(x_ref, o_ref):
    a = x_ref[:, 1:, :]
    rows = lax.broadcasted_iota(jnp.int32, a.shape, 1)
    work = a
    theta = None
    for _ in range(MAXP):
        mj = jnp.max(work, axis=1, keepdims=True)
        hit = work == mj
        r = jnp.min(jnp.where(hit, rows, F), axis=1, keepdims=True)
        work = jnp.where(rows == r, -jnp.inf, work)
        theta = mj
    ok = (a >= theta) & (a > PWR)
    f0 = jnp.min(jnp.where(ok, rows + 1, F), axis=1, keepdims=True)
    f0 = jnp.where(f0 == F, 0, f0)
    f0f = f0.astype(jnp.float32)
    safe = jnp.maximum(f0f, 1.0)
    kk = lax.broadcasted_iota(jnp.int32, (B, F, T), 1).astype(jnp.float32)
    mmax = jnp.floor(jnp.float32(LLIM - 1) / safe)
    m = jnp.minimum(mmax, jnp.floor((kk + MARGIN) / safe))
    d = jnp.abs(kk - m * f0f)
    cover = (f0f > 0.0) & (m >= 1.0) & (d <= MARGIN)
    val = jnp.maximum(1.0 - d * (0.5 / MARGIN), 0.5)
    o_ref[...] = jnp.where(cover, val, jnp.float32(0.5))


@functools.partial(jax.jit, static_argnames=("interpret",))
def _tc_mask(x2, interpret=False):
    return pl.pallas_call(
        _tc_body,
        out_shape=jax.ShapeDtypeStruct((B, F, T), jnp.float32),
        interpret=interpret,
    )(x2)


LANES = 16


def _sc_body(x_hbm, o_hbm, in_v, out_v, sem):
    b = lax.axis_index("c")
    t0 = lax.axis_index("s") * LANES
    pltpu.async_copy(x_hbm.at[b, :, pl.ds(t0, LANES)], in_v, sem).wait()

    neg = jnp.full((LANES,), -jnp.inf, jnp.float32)

    def p1(k, ms):
        v = in_v[k]
        out = []
        for mj in ms:
            out.append(jnp.maximum(mj, v))
            v = jnp.minimum(mj, v)
        return tuple(out)

    theta = lax.fori_loop(1, F, p1, (neg,) * MAXP)[MAXP - 1]

    def p2(k, best):
        v = in_v[k]
        cond = (v >= theta) & (v > PWR)
        return jnp.minimum(best, jnp.where(cond, jnp.full((LANES,), k), F))

    best = lax.fori_loop(1, F, p2, jnp.full((LANES,), F, jnp.int32))
    f0i = jnp.where(best == F, 0, best)
    f0f = f0i.astype(jnp.float32)
    safe = jnp.maximum(f0f, 1.0)
    mmax = (jnp.float32(LLIM - 1) / safe).astype(jnp.int32)
    covered = f0i > 0

    @pl.loop(0, F)
    def p3(k):
        kv = jnp.full((LANES,), k, jnp.int32)
        kf = kv.astype(jnp.float32)
        m = jnp.minimum(mmax, ((kf + MARGIN) / safe).astype(jnp.int32))
        d = jnp.abs(kv - m * f0i)
        cover = covered & (m >= 1) & (d <= MARGIN)
        val = jnp.maximum(1.0 - d.astype(jnp.float32) * (0.5 / MARGIN), 0.5)
        out_v[k] = jnp.where(cover, val, jnp.float32(0.5))

    pltpu.async_copy(out_v, o_hbm.at[b, :, pl.ds(t0, LANES)], sem).wait()


@jax.jit
def _sc_mask(x2):
    kern = pl.kernel(
        _sc_body,
        out_type=jax.ShapeDtypeStruct((B, F, T), jnp.float32),
        mesh=plsc.VectorSubcoreMesh(core_axis_name="c", subcore_axis_name="s"),
        compiler_params=pltpu.CompilerParams(use_tc_tiling_on_sc=False),
        scratch_types=[
            pltpu.VMEM((F, LANES), jnp.float32),
            pltpu.VMEM((F, LANES), jnp.float32),
            pltpu.SemaphoreType.DMA,
        ],
    )
    return kern(x2)


def kernel(x):
    x2 = x.reshape(B, F, T)
    return _sc_mask(x2).reshape(B, 1, F, T)
